# Initial kernel scaffold; baseline (speedup 1.0000x reference)
import jax
import jax.numpy as jnp
from jax.experimental import pallas as pl


def kernel(input_ids, position_ids, token_type_ids, word_embeddings, position_embeddings, token_type_embeddings):
    raise NotImplementedError("write your pallas kernel here")


# speedup vs baseline: 8.0092x; 8.0092x over previous
import functools

import jax
import jax.numpy as jnp
from jax import lax
from jax.experimental import pallas as pl
from jax.experimental.pallas import tpu as pltpu
from jax.experimental.pallas import tpu_sc as plsc

NC = 2
NS = 16
L = 16
EMBED = 128
BLK = 128


def _fuse_tables_body(typ_ref, pos_ref, out_ref):
    p = pos_ref[...]
    t = typ_ref[...]
    out_ref[...] = t[:, None, :] + p[None, :, :]


def _fuse_tables(type_emb, pos_emb):
    tv, e = type_emb.shape
    mp, _ = pos_emb.shape
    out = pl.pallas_call(
        _fuse_tables_body,
        out_shape=jax.ShapeDtypeStruct((tv, mp, e), jnp.float32),
    )(type_emb, pos_emb)
    return out.reshape(tv * mp, e)


def _sc_body(nblk, max_pos, ids_hbm, pid_hbm, tt_hbm, word_hbm, fused_hbm,
             out_hbm, ids_v, pid_v, tt_v, fidx_v, wbuf, pbuf, sem_w, sem_p):
    wid = lax.axis_index("s") * NC + lax.axis_index("c")
    base = wid * (nblk * BLK)

    def blk(i, carry):
        t0 = base + i * BLK
        pltpu.sync_copy(ids_hbm.at[pl.ds(t0, BLK)], ids_v)
        pltpu.sync_copy(pid_hbm.at[pl.ds(t0, BLK)], pid_v)
        pltpu.sync_copy(tt_hbm.at[pl.ds(t0, BLK)], tt_v)
        cw = pltpu.async_copy(word_hbm.at[ids_v], wbuf, sem_w)

        def fk(k, c2):
            sl = pl.ds(k * L, L)
            fidx_v[sl] = tt_v[sl] * max_pos + pid_v[sl]
            return c2

        lax.fori_loop(0, BLK // L, fk, 0)
        cp = pltpu.async_copy(fused_hbm.at[fidx_v], pbuf, sem_p)
        cw.wait()
        cp.wait()

        def ak(j, c2):
            for c in range(EMBED // L):
                sl = pl.ds(c * L, L)
                plsc.addupdate(wbuf.at[j, sl], pbuf[j, sl])
            return c2

        lax.fori_loop(0, BLK, ak, 0)
        pltpu.sync_copy(wbuf, out_hbm.at[pl.ds(t0, BLK)])
        return carry

    lax.fori_loop(0, nblk, blk, 0)


def kernel(input_ids, position_ids, token_type_ids, word_embeddings,
           position_embeddings, token_type_embeddings):
    batch, seqlen = input_ids.shape
    tok = batch * seqlen
    nw = NC * NS
    per_w = tok // nw
    nblk = per_w // BLK
    max_pos = position_embeddings.shape[0]

    ids = input_ids.reshape(-1).astype(jnp.int32)
    pid = position_ids.reshape(-1).astype(jnp.int32)
    tt = token_type_ids.reshape(-1).astype(jnp.int32)

    fused = _fuse_tables(token_type_embeddings, position_embeddings)

    mesh = plsc.VectorSubcoreMesh(core_axis_name="c", subcore_axis_name="s")
    sc = pl.kernel(
        functools.partial(_sc_body, nblk, max_pos),
        out_type=jax.ShapeDtypeStruct((tok, EMBED), jnp.float32),
        mesh=mesh,
        scratch_types=[
            pltpu.VMEM((BLK,), jnp.int32),
            pltpu.VMEM((BLK,), jnp.int32),
            pltpu.VMEM((BLK,), jnp.int32),
            pltpu.VMEM((BLK,), jnp.int32),
            pltpu.VMEM((BLK, EMBED), jnp.float32),
            pltpu.VMEM((BLK, EMBED), jnp.float32),
            pltpu.SemaphoreType.DMA,
            pltpu.SemaphoreType.DMA,
        ],
    )
    out = sc(ids, pid, tt, word_embeddings, fused)
    return out.reshape(batch, seqlen, EMBED)


# speedup vs baseline: 14.0899x; 1.7592x over previous
import functools

import jax
import jax.numpy as jnp
from jax import lax
from jax.experimental import pallas as pl
from jax.experimental.pallas import tpu as pltpu
from jax.experimental.pallas import tpu_sc as plsc

NC = 2
NS = 16
L = 16
EMBED = 128
BLK = 64
NBUF = 4


def _fuse_tables_body(typ_ref, pos_ref, out_ref):
    p = pos_ref[...]
    t = typ_ref[...]
    out_ref[...] = t[:, None, :] + p[None, :, :]


def _fuse_tables(type_emb, pos_emb):
    tv, e = type_emb.shape
    mp, _ = pos_emb.shape
    out = pl.pallas_call(
        _fuse_tables_body,
        out_shape=jax.ShapeDtypeStruct((tv, mp, e), jnp.float32),
    )(type_emb, pos_emb)
    return out.reshape(tv * mp, e)


def _sc_body(nblk, max_pos, ids_hbm, pid_hbm, tt_hbm, word_hbm, fused_hbm,
             out_hbm, ids_v, pid_v, tt_v, fidx_v, wbuf, pbuf, sem_i,
             sem_w0, sem_w1, sem_w2, sem_w3,
             sem_p0, sem_p1, sem_p2, sem_p3,
             sem_o0, sem_o1, sem_o2, sem_o3):
    sems_w = (sem_w0, sem_w1, sem_w2, sem_w3)
    sems_p = (sem_p0, sem_p1, sem_p2, sem_p3)
    sems_o = (sem_o0, sem_o1, sem_o2, sem_o3)
    wid = lax.axis_index("s") * NC + lax.axis_index("c")
    base = wid * (nblk * BLK)

    def issue(g, s):
        t0 = base + g * BLK
        c1 = pltpu.async_copy(ids_hbm.at[pl.ds(t0, BLK)], ids_v.at[s], sem_i)
        c2 = pltpu.async_copy(pid_hbm.at[pl.ds(t0, BLK)], pid_v.at[s], sem_i)
        c3 = pltpu.async_copy(tt_hbm.at[pl.ds(t0, BLK)], tt_v.at[s], sem_i)
        c1.wait()
        c2.wait()
        c3.wait()
        for k in range(BLK // L):
            sl = pl.ds(k * L, L)
            fidx_v[s, sl] = tt_v[s, sl] * max_pos + pid_v[s, sl]
        pltpu.async_copy(word_hbm.at[ids_v.at[s]], wbuf.at[s], sems_w[s])
        pltpu.async_copy(fused_hbm.at[fidx_v.at[s]], pbuf.at[s], sems_p[s])

    def wait_gathers(s):
        pltpu.make_async_copy(word_hbm.at[ids_v.at[s]], wbuf.at[s],
                              sems_w[s]).wait()
        pltpu.make_async_copy(fused_hbm.at[fidx_v.at[s]], pbuf.at[s],
                              sems_p[s]).wait()

    def wait_out(s):
        pltpu.make_async_copy(pbuf.at[s], out_hbm.at[pl.ds(base, BLK)],
                              sems_o[s]).wait()

    def add_and_store(g, s):
        def ak(j, c2):
            for c in range(EMBED // L):
                sl = pl.ds(c * L, L)
                plsc.addupdate(pbuf.at[s, j, sl], wbuf[s, j, sl])
            return c2

        lax.fori_loop(0, BLK, ak, 0)
        t0 = base + g * BLK
        pltpu.async_copy(pbuf.at[s], out_hbm.at[pl.ds(t0, BLK)], sems_o[s])

    nout = nblk // NBUF
    issue(0, 0)

    def outer(g0, carry):
        for b in range(NBUF):
            g = g0 * NBUF + b
            s = b
            ns = (b + 1) % NBUF
            if b < NBUF - 1:
                @pl.when(g0 >= 1)
                def _():
                    wait_out(ns)
                issue(g + 1, ns)
            else:
                @pl.when(g0 < nout - 1)
                def _():
                    wait_out(ns)
                    issue(g + 1, ns)
            wait_gathers(s)
            add_and_store(g, s)
        return carry

    lax.fori_loop(0, nout, outer, 0)
    for s in range(NBUF):
        wait_out(s)


def kernel(input_ids, position_ids, token_type_ids, word_embeddings,
           position_embeddings, token_type_embeddings):
    batch, seqlen = input_ids.shape
    tok = batch * seqlen
    nw = NC * NS
    per_w = tok // nw
    nblk = per_w // BLK
    max_pos = position_embeddings.shape[0]

    ids = input_ids.reshape(-1).astype(jnp.int32)
    pid = position_ids.reshape(-1).astype(jnp.int32)
    tt = token_type_ids.reshape(-1).astype(jnp.int32)

    fused = _fuse_tables(token_type_embeddings, position_embeddings)

    mesh = plsc.VectorSubcoreMesh(core_axis_name="c", subcore_axis_name="s")
    sc = pl.kernel(
        functools.partial(_sc_body, nblk, max_pos),
        out_type=jax.ShapeDtypeStruct((tok, EMBED), jnp.float32),
        mesh=mesh,
        scratch_types=[
            pltpu.VMEM((NBUF, BLK), jnp.int32),
            pltpu.VMEM((NBUF, BLK), jnp.int32),
            pltpu.VMEM((NBUF, BLK), jnp.int32),
            pltpu.VMEM((NBUF, BLK), jnp.int32),
            pltpu.VMEM((NBUF, BLK, EMBED), jnp.float32),
            pltpu.VMEM((NBUF, BLK, EMBED), jnp.float32),
        ] + [pltpu.SemaphoreType.DMA] * 13,
    )
    out = sc(ids, pid, tt, word_embeddings, fused)
    return out.reshape(batch, seqlen, EMBED)
